# baseline (speedup 1.0000x reference)
import functools

import jax
import jax.numpy as jnp
from jax import lax
from jax.experimental import pallas as pl
from jax.experimental.pallas import tpu as pltpu
from jax.experimental.pallas import tpu_sc as plsc

B = 16
C = 32
P = 1024
K = 1024
N = B * P
NW = 32
CHUNK = N // NW
LOG2E = 1.4426950408889634
LN2 = 0.6931471805599453


def _vq_main_body(pq_ref, z_ref, cb2_ref, lhs_ref,
                  zqt_ref, idx_ref, plogp_ref, negm_ref):
    b = pl.program_id(0)
    w = 0.5 / jnp.maximum(pq_ref[0, 0], 1e-10)
    z = z_ref[0]
    cb2 = cb2_ref[...]
    lhs = lhs_ref[...]

    zsq = jnp.sum(z * z, axis=0, keepdims=True)
    csq = 0.25 * jnp.sum(cb2 * cb2, axis=1, keepdims=True)
    g2 = jnp.dot(cb2, z, preferred_element_type=jnp.float32)
    dist = (zsq + csq) - g2
    lam = w * dist

    m2 = jnp.min(lam, axis=0, keepdims=True)
    e = jnp.where(lam == m2, 1.0, 0.0).astype(jnp.bfloat16)
    r = jnp.dot(lhs, e, preferred_element_type=jnp.float32)
    zqt_ref[0] = r[:C]
    idxf = r[C] * 32.0 + r[C + 1]
    idx_ref[0] = jnp.clip(idxf.astype(jnp.int32), 0, K - 1)[None, :]

    u2 = (m2 - lam) * LOG2E
    eu = jnp.exp2(u2)
    ones_row = jnp.ones((1, K), jnp.bfloat16)
    eu_bf = eu.astype(jnp.bfloat16)
    s = jnp.dot(ones_row, eu_bf,
                preferred_element_type=jnp.float32)
    t2 = jnp.dot(ones_row, u2.astype(jnp.bfloat16) * eu_bf,
                 preferred_element_type=jnp.float32)
    plogp = (t2 * LN2) / s - jnp.log(s)

    @pl.when(b == 0)
    def _():
        plogp_ref[...] = jnp.zeros_like(plogp_ref)
        negm_ref[...] = jnp.zeros_like(negm_ref)

    plogp_ref[...] += plogp
    negm_ref[...] += m2


def _vq_main(pq, z, cb2, lhs):
    return pl.pallas_call(
        _vq_main_body,
        grid=(B,),
        in_specs=[
            pl.BlockSpec(memory_space=pltpu.SMEM),
            pl.BlockSpec((1, C, P), lambda b: (b, 0, 0)),
            pl.BlockSpec((K, C), lambda b: (0, 0)),
            pl.BlockSpec((C + 2, K), lambda b: (0, 0)),
        ],
        out_specs=[
            pl.BlockSpec((1, C, P), lambda b: (b, 0, 0)),
            pl.BlockSpec((1, 1, P), lambda b: (b, 0, 0)),
            pl.BlockSpec((1, P), lambda b: (0, 0)),
            pl.BlockSpec((1, P), lambda b: (0, 0)),
        ],
        out_shape=[
            jax.ShapeDtypeStruct((B, C, P), jnp.float32),
            jax.ShapeDtypeStruct((B, 1, P), jnp.int32),
            jax.ShapeDtypeStruct((1, P), jnp.float32),
            jax.ShapeDtypeStruct((1, P), jnp.float32),
        ],
    )(pq, z, cb2, lhs)


def _sc_hist_body(idx_hbm, out_hbm, idx_v, hist_v):
    cc = lax.axis_index("c")
    ss = lax.axis_index("s")
    wid = ss * 2 + cc
    pltpu.sync_copy(idx_hbm.at[wid], idx_v)

    zeros16 = jnp.zeros((16,), jnp.float32)

    def zero_body(i, carry):
        hist_v[pl.ds(i * 16, 16)] = zeros16
        return carry

    lax.fori_loop(0, (16 * K) // 16, zero_body, 0)

    lane_off = lax.iota(jnp.int32, 16) * K
    ones = jnp.ones((16,), jnp.float32)

    def body(i, carry):
        v = idx_v[pl.ds(i * 16, 16)]
        plsc.addupdate_scatter(hist_v, [lane_off + v], ones)
        return carry

    lax.fori_loop(0, CHUNK // 16, body, 0)

    def red_body(i, carry):
        acc = hist_v[pl.ds(i * 16, 16)]
        for l in range(1, 16):
            acc = acc + hist_v[pl.ds(l * K + i * 16, 16)]
        hist_v[pl.ds(i * 16, 16)] = acc
        return carry

    lax.fori_loop(0, K // 16, red_body, 0)
    pltpu.sync_copy(hist_v.at[pl.ds(0, K)], out_hbm.at[wid])


@functools.cache
def _sc_hist_kernel():
    return pl.kernel(
        _sc_hist_body,
        out_type=jax.ShapeDtypeStruct((NW, K), jnp.float32),
        mesh=plsc.VectorSubcoreMesh(
            core_axis_name="c", subcore_axis_name="s", num_cores=2),
        scratch_types=[
            pltpu.VMEM((CHUNK,), jnp.int32),
            pltpu.VMEM((16 * K,), jnp.float32),
        ],
        compiler_params=pltpu.CompilerParams(needs_layout_passes=False),
    )


def _finish_body(sub_ref, plogp_ref, negm_ref, loss_ref, perp_ref):
    counts = jnp.sum(sub_ref[...], axis=0, keepdims=True)
    avg = counts * (1.0 / N)
    ent = jnp.sum(avg * jnp.log(avg + 1e-7))
    perp_ref[...] = jnp.zeros_like(perp_ref) + jnp.exp(-ent)
    tot = jnp.sum(plogp_ref[...]) + jnp.sum(negm_ref[...])
    loss_ref[...] = jnp.zeros_like(loss_ref) + tot * (1.0 / B)


def _finish(sub, plogp, negm):
    return pl.pallas_call(
        _finish_body,
        out_shape=[
            jax.ShapeDtypeStruct((1, 128), jnp.float32),
            jax.ShapeDtypeStruct((1, 128), jnp.float32),
        ],
    )(sub, plogp, negm)


def kernel(z_from_encoder, param_q, codebook, flg_train, flg_quant_det):
    z = z_from_encoder.reshape(B, C, P)
    pq = param_q.reshape(1, 1)
    kr = jnp.arange(K, dtype=jnp.int32)
    lhs = jnp.concatenate(
        [codebook.T.astype(jnp.bfloat16),
         (kr // 32).astype(jnp.bfloat16)[None, :],
         (kr % 32).astype(jnp.bfloat16)[None, :]], axis=0)
    zqt, idx, plogp, negm = _vq_main(pq, z, codebook * 2.0, lhs)
    sub = _sc_hist_kernel()(idx.reshape(NW, CHUNK))
    loss, perp = _finish(sub, plogp, negm)
    return (zqt.reshape(B, C, 32, 32),
            loss[0, 0].reshape(()),
            perp[0, 0].reshape(()))


# speedup vs baseline: 1.0288x; 1.0288x over previous
import functools

import jax
import jax.numpy as jnp
from jax import lax
from jax.experimental import pallas as pl
from jax.experimental.pallas import tpu as pltpu
from jax.experimental.pallas import tpu_sc as plsc

B = 16
C = 32
P = 1024
K = 1024
N = B * P
NW = 32
CHUNK = N // NW
LOG2E = 1.4426950408889634
LN2 = 0.6931471805599453


def _vq_main_body(pq_ref, z_ref, cb2_ref, lhs_ref,
                  zqt_ref, idx_ref, plogp_ref, negm_ref):
    b = pl.program_id(0)
    w = 0.5 / jnp.maximum(pq_ref[0, 0], 1e-10)
    z = z_ref[0]
    cb2 = cb2_ref[...]
    lhs = lhs_ref[...]

    zsq = jnp.sum(z * z, axis=0, keepdims=True)
    csq = 0.25 * jnp.sum(cb2 * cb2, axis=1, keepdims=True)
    g2 = jnp.dot(cb2, z, preferred_element_type=jnp.float32)
    dist = (zsq + csq) - g2
    lam = w * dist

    m2 = jnp.min(lam, axis=0, keepdims=True)
    e = jnp.where(lam == m2, 1.0, 0.0).astype(jnp.bfloat16)
    r = jnp.dot(lhs, e, preferred_element_type=jnp.float32)
    idxf = r[0] * 32.0 + r[1]
    idx_ref[0] = jnp.clip(idxf.astype(jnp.int32), 0, K - 1)[None, :]
    zqt_ref[0] = r[8:8 + C]

    u2 = (m2 - lam).astype(jnp.bfloat16) * jnp.bfloat16(LOG2E)
    eu = jnp.exp2(u2)
    ones_row = jnp.ones((1, K), jnp.bfloat16)
    s = jnp.dot(ones_row, eu,
                preferred_element_type=jnp.float32)
    t2 = jnp.dot(ones_row, u2 * eu,
                 preferred_element_type=jnp.float32)
    plogp = (t2 * LN2) / s - jnp.log(s)

    @pl.when(b == 0)
    def _():
        plogp_ref[...] = jnp.zeros_like(plogp_ref)
        negm_ref[...] = jnp.zeros_like(negm_ref)

    plogp_ref[...] += plogp
    negm_ref[...] += m2


def _vq_main(pq, z, cb2, lhs):
    return pl.pallas_call(
        _vq_main_body,
        grid=(B,),
        in_specs=[
            pl.BlockSpec(memory_space=pltpu.SMEM),
            pl.BlockSpec((1, C, P), lambda b: (b, 0, 0)),
            pl.BlockSpec((K, C), lambda b: (0, 0)),
            pl.BlockSpec((40, K), lambda b: (0, 0)),
        ],
        out_specs=[
            pl.BlockSpec((1, C, P), lambda b: (b, 0, 0)),
            pl.BlockSpec((1, 1, P), lambda b: (b, 0, 0)),
            pl.BlockSpec((1, P), lambda b: (0, 0)),
            pl.BlockSpec((1, P), lambda b: (0, 0)),
        ],
        out_shape=[
            jax.ShapeDtypeStruct((B, C, P), jnp.float32),
            jax.ShapeDtypeStruct((B, 1, P), jnp.int32),
            jax.ShapeDtypeStruct((1, P), jnp.float32),
            jax.ShapeDtypeStruct((1, P), jnp.float32),
        ],
    )(pq, z, cb2, lhs)


def _sc_hist_body(idx_hbm, out_hbm, idx_v, hist_v):
    cc = lax.axis_index("c")
    ss = lax.axis_index("s")
    wid = ss * 2 + cc
    pltpu.sync_copy(idx_hbm.at[wid], idx_v)

    zeros16 = jnp.zeros((16,), jnp.float32)

    def zero_body(i, carry):
        hist_v[pl.ds(i * 16, 16)] = zeros16
        return carry

    lax.fori_loop(0, (16 * K) // 16, zero_body, 0)

    lane_off = lax.iota(jnp.int32, 16) * K
    ones = jnp.ones((16,), jnp.float32)

    def body(i, carry):
        v = idx_v[pl.ds(i * 16, 16)]
        plsc.addupdate_scatter(hist_v, [lane_off + v], ones)
        return carry

    lax.fori_loop(0, CHUNK // 16, body, 0)

    def red_body(i, carry):
        acc = hist_v[pl.ds(i * 16, 16)]
        for l in range(1, 16):
            acc = acc + hist_v[pl.ds(l * K + i * 16, 16)]
        hist_v[pl.ds(i * 16, 16)] = acc
        return carry

    lax.fori_loop(0, K // 16, red_body, 0)
    pltpu.sync_copy(hist_v.at[pl.ds(0, K)], out_hbm.at[wid])


@functools.cache
def _sc_hist_kernel():
    return pl.kernel(
        _sc_hist_body,
        out_type=jax.ShapeDtypeStruct((NW, K), jnp.float32),
        mesh=plsc.VectorSubcoreMesh(
            core_axis_name="c", subcore_axis_name="s", num_cores=2),
        scratch_types=[
            pltpu.VMEM((CHUNK,), jnp.int32),
            pltpu.VMEM((16 * K,), jnp.float32),
        ],
        compiler_params=pltpu.CompilerParams(needs_layout_passes=False),
    )


def _finish_body(sub_ref, plogp_ref, negm_ref, loss_ref, perp_ref):
    counts = jnp.sum(sub_ref[...], axis=0, keepdims=True)
    avg = counts * (1.0 / N)
    ent = jnp.sum(avg * jnp.log(avg + 1e-7))
    perp_ref[...] = jnp.zeros_like(perp_ref) + jnp.exp(-ent)
    tot = jnp.sum(plogp_ref[...]) + jnp.sum(negm_ref[...])
    loss_ref[...] = jnp.zeros_like(loss_ref) + tot * (1.0 / B)


def _finish(sub, plogp, negm):
    return pl.pallas_call(
        _finish_body,
        out_shape=[
            jax.ShapeDtypeStruct((1, 128), jnp.float32),
            jax.ShapeDtypeStruct((1, 128), jnp.float32),
        ],
    )(sub, plogp, negm)


def kernel(z_from_encoder, param_q, codebook, flg_train, flg_quant_det):
    z = z_from_encoder.reshape(B, C, P)
    pq = param_q.reshape(1, 1)
    kr = jnp.arange(K, dtype=jnp.int32)
    lhs = jnp.concatenate(
        [(kr // 32).astype(jnp.bfloat16)[None, :],
         (kr % 32).astype(jnp.bfloat16)[None, :],
         jnp.zeros((6, K), jnp.bfloat16),
         codebook.T.astype(jnp.bfloat16)], axis=0)
    zqt, idx, plogp, negm = _vq_main(pq, z, codebook * 2.0, lhs)
    sub = _sc_hist_kernel()(idx.reshape(NW, CHUNK))
    loss, perp = _finish(sub, plogp, negm)
    return (zqt.reshape(B, C, 32, 32),
            loss[0, 0].reshape(()),
            perp[0, 0].reshape(()))
